# baseline (speedup 1.0000x reference)
import functools

import jax
import jax.numpy as jnp
from jax import lax
from jax.experimental import pallas as pl
from jax.experimental.pallas import tpu as pltpu
from jax.experimental.pallas import tpu_sc as plsc

B, C, H, W = 2, 256, 64, 64
N = 5000
AH = AW = 7
SCALE = 0.0625
NPTS = N * AH * AW
NC, NS = 2, 16
NWORK = NC * NS
CH = 64
CPW = -(-NPTS // (NWORK * CH))
PPW = CPW * CH
NP_PAD = NWORK * PPW
LANES = 16


def _sc_body(ft_hbm, rois_hbm, out_hbm,
             rois_v, idx0, idx1, idx2, idx3, wb0, wb1, wb2, wb3,
             rows0, rows1, rows2, rows3, outb, sem):
    wid = lax.axis_index("s") * NC + lax.axis_index("c")
    pltpu.sync_copy(rois_hbm, rois_v)
    base = wid * PPW
    lane = lax.broadcasted_iota(jnp.int32, (LANES,), 0)

    def chunk_body(ci, carry):
        p0 = base + ci * CH
        for g in range(CH // LANES):
            p = lane + (p0 + g * LANES)
            n = lax.div(p, 49)
            rem = p - n * 49
            ph = lax.div(rem, 7)
            pw = rem - ph * 7
            n = jnp.minimum(n, N - 1)
            i5 = n * 5
            bf = plsc.load_gather(rois_v, [i5])
            x1 = plsc.load_gather(rois_v, [i5 + 1]) * SCALE
            y1 = plsc.load_gather(rois_v, [i5 + 2]) * SCALE
            x2 = plsc.load_gather(rois_v, [i5 + 3]) * SCALE
            y2 = plsc.load_gather(rois_v, [i5 + 4]) * SCALE
            bw = jnp.maximum(x2 - x1, 0.0) * (1.0 / (AW - 1))
            bh = jnp.maximum(y2 - y1, 0.0) * (1.0 / (AH - 1))
            hf = y1 + ph.astype(jnp.float32) * bh
            wf = x1 + pw.astype(jnp.float32) * bw
            valid = (hf >= 0.0) & (hf < float(H)) & (wf >= 0.0) & (wf < float(W))
            h0 = jnp.clip(hf, 0.0, float(H - 1)).astype(jnp.int32)
            w0 = jnp.clip(wf, 0.0, float(W - 1)).astype(jnp.int32)
            lh = hf - h0.astype(jnp.float32)
            lw = wf - w0.astype(jnp.float32)
            h1 = jnp.minimum(h0 + 1, H - 1)
            w1 = jnp.minimum(w0 + 1, W - 1)
            rowb = bf.astype(jnp.int32) * (H * W)
            r0 = rowb + h0 * W
            r1 = rowb + h1 * W
            vf = jnp.where(valid, 1.0, 0.0).astype(jnp.float32)
            olh = (1.0 - lh) * vf
            olw = 1.0 - lw
            sl = pl.ds(g * LANES, LANES)
            idx0[sl] = r0 + w0
            idx1[sl] = r0 + w1
            idx2[sl] = r1 + w0
            idx3[sl] = r1 + w1
            wb0[sl] = olh * olw
            wb1[sl] = olh * lw
            wb2[sl] = lh * vf * olw
            wb3[sl] = lh * vf * lw
        d0 = pltpu.async_copy(ft_hbm.at[idx0], rows0, sem)
        d1 = pltpu.async_copy(ft_hbm.at[idx1], rows1, sem)
        d2 = pltpu.async_copy(ft_hbm.at[idx2], rows2, sem)
        d3 = pltpu.async_copy(ft_hbm.at[idx3], rows3, sem)
        d0.wait()
        d1.wait()
        d2.wait()
        d3.wait()

        def point_body(j, jcarry):
            jj = jnp.zeros((LANES,), jnp.int32) + j
            wv0 = plsc.load_gather(wb0, [jj])
            wv1 = plsc.load_gather(wb1, [jj])
            wv2 = plsc.load_gather(wb2, [jj])
            wv3 = plsc.load_gather(wb3, [jj])
            for cb in range(C // LANES):
                cs = pl.ds(cb * LANES, LANES)
                acc = (wv0 * rows0[j, cs] + wv1 * rows1[j, cs]
                       + wv2 * rows2[j, cs] + wv3 * rows3[j, cs])
                outb[j, cs] = acc
            return jcarry

        lax.fori_loop(0, CH, point_body, 0)
        pltpu.sync_copy(outb, out_hbm.at[pl.ds(p0, CH)])
        return carry

    lax.fori_loop(0, CPW, chunk_body, 0)


PB = AH * AW
TG = 8


def _tc_transpose_body(x_ref, o_ref):
    x = x_ref[...]
    eye = jnp.eye(PB, dtype=jnp.float32)
    for g in range(TG):
        xg = x[g * PB:(g + 1) * PB, :]
        o_ref[g] = lax.dot_general(xg, eye, (((0,), (0,)), ((), ())),
                                   preferred_element_type=jnp.float32)


@jax.jit
def kernel(features, rois):
    ft = jnp.transpose(features, (0, 2, 3, 1)).reshape(B * H * W, C)
    rois_flat = rois.reshape(-1)
    mesh = plsc.VectorSubcoreMesh(core_axis_name="c", subcore_axis_name="s",
                                  num_cores=NC, num_subcores=NS)
    out = pl.kernel(
        _sc_body,
        out_type=jax.ShapeDtypeStruct((NP_PAD, C), jnp.float32),
        mesh=mesh,
        compiler_params=pltpu.CompilerParams(needs_layout_passes=False),
        scratch_types=[
            pltpu.VMEM((N * 5,), jnp.float32),
            pltpu.VMEM((CH,), jnp.int32),
            pltpu.VMEM((CH,), jnp.int32),
            pltpu.VMEM((CH,), jnp.int32),
            pltpu.VMEM((CH,), jnp.int32),
            pltpu.VMEM((CH,), jnp.float32),
            pltpu.VMEM((CH,), jnp.float32),
            pltpu.VMEM((CH,), jnp.float32),
            pltpu.VMEM((CH,), jnp.float32),
            pltpu.VMEM((CH, C), jnp.float32),
            pltpu.VMEM((CH, C), jnp.float32),
            pltpu.VMEM((CH, C), jnp.float32),
            pltpu.VMEM((CH, C), jnp.float32),
            pltpu.VMEM((CH, C), jnp.float32),
            pltpu.SemaphoreType.DMA,
        ],
    )(ft, rois_flat)
    outt = pl.pallas_call(
        _tc_transpose_body,
        grid=(N // TG,),
        in_specs=[pl.BlockSpec((TG * PB, C), lambda i: (i, 0))],
        out_specs=pl.BlockSpec((TG, C, PB), lambda i: (i, 0, 0)),
        out_shape=jax.ShapeDtypeStruct((N, C, PB), jnp.float32),
    )(out)
    return outt.reshape(N, C, AH, AW)


# speedup vs baseline: 1.2273x; 1.2273x over previous
import jax
import jax.numpy as jnp
from jax import lax
from jax.experimental import pallas as pl
from jax.experimental.pallas import tpu as pltpu
from jax.experimental.pallas import tpu_sc as plsc

B, C, H, W = 2, 256, 64, 64
N = 5000
AH = AW = 7
PB = AH * AW
SCALE = 0.0625
NPTS = N * PB
NC, NS = 2, 16
NWORK = NC * NS
CH = 32
CPW = -(-NPTS // (NWORK * CH))
PPW = CPW * CH
NP_PAD = NWORK * PPW
LANES = 16
TG = 40


def _sc_body(ft_hbm, rois_hbm, out_hbm, rois_v, *s):
    slot_a, slot_b = list(s[0:13]), list(s[13:26])
    sem_a, sem_b = s[26], s[27]
    wid = lax.axis_index("s") * NC + lax.axis_index("c")
    pltpu.sync_copy(rois_hbm, rois_v)
    base = wid * PPW
    lane = lax.broadcasted_iota(jnp.int32, (LANES,), 0)

    def fire(ci, slot, sem):
        idxs, wbs, rows = slot[0:4], slot[4:8], slot[8:12]
        p0 = base + ci * CH
        for g in range(CH // LANES):
            p = lane + (p0 + g * LANES)
            n = lax.div(p, PB)
            rem = p - n * PB
            ph = lax.div(rem, AW)
            pw = rem - ph * AW
            n = jnp.minimum(n, N - 1)
            i5 = n * 5
            bf = plsc.load_gather(rois_v, [i5])
            x1 = plsc.load_gather(rois_v, [i5 + 1]) * SCALE
            y1 = plsc.load_gather(rois_v, [i5 + 2]) * SCALE
            x2 = plsc.load_gather(rois_v, [i5 + 3]) * SCALE
            y2 = plsc.load_gather(rois_v, [i5 + 4]) * SCALE
            bw = jnp.maximum(x2 - x1, 0.0) * (1.0 / (AW - 1))
            bh = jnp.maximum(y2 - y1, 0.0) * (1.0 / (AH - 1))
            hf = y1 + ph.astype(jnp.float32) * bh
            wf = x1 + pw.astype(jnp.float32) * bw
            valid = (hf >= 0.0) & (hf < float(H)) & (wf >= 0.0) & (wf < float(W))
            h0 = jnp.clip(hf, 0.0, float(H - 1)).astype(jnp.int32)
            w0 = jnp.clip(wf, 0.0, float(W - 1)).astype(jnp.int32)
            lh = hf - h0.astype(jnp.float32)
            lw = wf - w0.astype(jnp.float32)
            h1 = jnp.minimum(h0 + 1, H - 1)
            w1 = jnp.minimum(w0 + 1, W - 1)
            rowb = bf.astype(jnp.int32) * (H * W)
            r0 = rowb + h0 * W
            r1 = rowb + h1 * W
            vf = jnp.where(valid, 1.0, 0.0).astype(jnp.float32)
            olh = (1.0 - lh) * vf
            olw = 1.0 - lw
            sl = pl.ds(g * LANES, LANES)
            idxs[0][sl] = r0 + w0
            idxs[1][sl] = r0 + w1
            idxs[2][sl] = r1 + w0
            idxs[3][sl] = r1 + w1
            wbs[0][sl] = olh * olw
            wbs[1][sl] = olh * lw
            wbs[2][sl] = lh * vf * olw
            wbs[3][sl] = lh * vf * lw
        for k in range(4):
            pltpu.async_copy(ft_hbm.at[idxs[k]], rows[k], sem)

    def drain(slot, sem):
        idxs, rows = slot[0:4], slot[8:12]
        for k in range(4):
            pltpu.make_async_copy(ft_hbm.at[idxs[k]], rows[k], sem).wait()

    def combine(ci, slot):
        wbs, rows, outb = slot[4:8], slot[8:12], slot[12]

        def point_body(j, jcarry):
            jj = jnp.zeros((LANES,), jnp.int32) + j
            wv0 = plsc.load_gather(wbs[0], [jj])
            wv1 = plsc.load_gather(wbs[1], [jj])
            wv2 = plsc.load_gather(wbs[2], [jj])
            wv3 = plsc.load_gather(wbs[3], [jj])
            for cb in range(C // LANES):
                cs = pl.ds(cb * LANES, LANES)
                acc = (wv0 * rows[0][j, cs] + wv1 * rows[1][j, cs]
                       + wv2 * rows[2][j, cs] + wv3 * rows[3][j, cs])
                outb[j, cs] = acc
            return jcarry

        lax.fori_loop(0, CH, point_body, 0)
        pltpu.sync_copy(outb, out_hbm.at[pl.ds(base + ci * CH, CH)])

    fire(0, slot_a, sem_a)

    def pair_body(k, carry):
        ci_a = 2 * k
        fire(ci_a + 1, slot_b, sem_b)
        drain(slot_a, sem_a)
        combine(ci_a, slot_a)
        fire(ci_a + 2, slot_a, sem_a)
        drain(slot_b, sem_b)
        combine(ci_a + 1, slot_b)
        return carry

    lax.fori_loop(0, CPW // 2, pair_body, 0)
    drain(slot_a, sem_a)


def _tc_transpose_body(x_ref, o_ref):
    x = x_ref[...]
    eye = jnp.eye(PB, dtype=jnp.float32)
    for g in range(TG):
        xg = x[g * PB:(g + 1) * PB, :]
        o_ref[g] = lax.dot_general(xg, eye, (((0,), (0,)), ((), ())),
                                   precision=lax.Precision.HIGHEST,
                                   preferred_element_type=jnp.float32)


@jax.jit
def kernel(features, rois):
    ft = jnp.transpose(features, (0, 2, 3, 1)).reshape(B * H * W, C)
    rois_flat = rois.reshape(-1)
    mesh = plsc.VectorSubcoreMesh(core_axis_name="c", subcore_axis_name="s",
                                  num_cores=NC, num_subcores=NS)
    slot = ([pltpu.VMEM((CH,), jnp.int32)] * 4
            + [pltpu.VMEM((CH,), jnp.float32)] * 4
            + [pltpu.VMEM((CH, C), jnp.float32)] * 5)
    out = pl.kernel(
        _sc_body,
        out_type=jax.ShapeDtypeStruct((NP_PAD, C), jnp.float32),
        mesh=mesh,
        compiler_params=pltpu.CompilerParams(needs_layout_passes=False),
        scratch_types=([pltpu.VMEM((N * 5,), jnp.float32)] + slot + slot
                       + [pltpu.SemaphoreType.DMA, pltpu.SemaphoreType.DMA]),
    )(ft, rois_flat)
    outt = pl.pallas_call(
        _tc_transpose_body,
        grid=(N // TG,),
        in_specs=[pl.BlockSpec((TG * PB, C), lambda i: (i, 0))],
        out_specs=pl.BlockSpec((TG, C, PB), lambda i: (i, 0, 0)),
        out_shape=jax.ShapeDtypeStruct((N, C, PB), jnp.float32),
    )(out)
    return outt.reshape(N, C, AH, AW)
